# baseline (speedup 1.0000x reference)
import functools
import math

import jax
import jax.numpy as jnp
from jax import lax
from jax.experimental import pallas as pl
from jax.experimental.pallas import tpu as pltpu
from jax.experimental.pallas import tpu_sc as plsc

EMB = 64
LANES = 16
IDXW = 128
K = 8
NC = 2
NS = 16
NW = NC * NS
SCALE = math.sqrt(EMB)


def _emb_body(idx_hbm, table_hbm, out_hbm, idx_v, rows_v, sem):
    wid = lax.axis_index("s") * NC + lax.axis_index("c")
    idx_rows = idx_hbm.shape[0] // NW
    chunks = idx_rows // K
    row0 = wid * idx_rows

    def chunk_body(ch, carry):
        base = row0 + ch * K
        pltpu.sync_copy(idx_hbm.at[pl.ds(base, K)], idx_v)
        copies = [
            pltpu.async_copy(
                table_hbm.at[idx_v.at[j]],
                rows_v.at[pl.ds(j * IDXW, IDXW)],
                sem,
            )
            for j in range(K)
        ]
        for cp in copies:
            cp.wait()

        def scale_row(r, c2):
            for c in range(EMB // LANES):
                sl = pl.ds(c * LANES, LANES)
                rows_v[r, sl] = rows_v[r, sl] * SCALE
            return c2

        lax.fori_loop(0, K * IDXW, scale_row, 0)
        pltpu.sync_copy(rows_v, out_hbm.at[pl.ds(base * IDXW, K * IDXW)])
        return carry

    lax.fori_loop(0, chunks, chunk_body, 0)


def kernel(tokens, table):
    b, t = tokens.shape
    n = b * t
    idx = tokens.reshape(n).astype(jnp.int32).reshape(n // IDXW, IDXW)
    mesh = plsc.VectorSubcoreMesh(core_axis_name="c", subcore_axis_name="s")
    run = functools.partial(
        pl.kernel,
        mesh=mesh,
        compiler_params=pltpu.CompilerParams(use_tc_tiling_on_sc=False),
        out_type=jax.ShapeDtypeStruct((n, EMB), jnp.float32),
        scratch_types=[
            pltpu.VMEM((K, IDXW), jnp.int32),
            pltpu.VMEM((K * IDXW, EMB), jnp.float32),
            pltpu.SemaphoreType.DMA,
        ],
    )(_emb_body)
    out = run(idx, table)
    return out.reshape(b, t, EMB)


# speedup vs baseline: 1.1080x; 1.1080x over previous
import functools
import math

import jax
import jax.numpy as jnp
from jax import lax
from jax.experimental import pallas as pl
from jax.experimental.pallas import tpu as pltpu
from jax.experimental.pallas import tpu_sc as plsc

EMB = 64
LANES = 16
IDXW = 128
K = 5
C = K * IDXW
NC = 2
NS = 16
NW = NC * NS
SCALE = math.sqrt(EMB)
UNROLL = 8


def _emb_body(idx_hbm, table_hbm, out_hbm,
              idx_v, rows0, rows1, sg0, sg1, sw0, sw1):
    wid = lax.axis_index("s") * NC + lax.axis_index("c")
    idx_rows = idx_hbm.shape[0] // NW
    chunks = idx_rows // K
    row0 = wid * idx_rows

    rows = (rows0, rows1)
    sg = (sg0, sg1)
    sw = (sw0, sw1)

    pltpu.sync_copy(idx_hbm.at[pl.ds(row0, idx_rows)], idx_v)

    def fire_gathers(ch, buf):
        for j in range(K):
            pltpu.async_copy(
                table_hbm.at[idx_v.at[ch * K + j]],
                rows[buf].at[pl.ds(j * IDXW, IDXW)],
                sg[buf],
            )

    def wait_gathers(buf):
        pltpu.make_async_copy(out_hbm.at[pl.ds(0, C)], rows[buf], sg[buf]).wait()

    def fire_wb(ch, buf):
        pltpu.async_copy(
            rows[buf], out_hbm.at[pl.ds((row0 + ch * K) * IDXW, C)], sw[buf])

    def wait_wb(buf):
        pltpu.make_async_copy(rows[buf], out_hbm.at[pl.ds(0, C)], sw[buf]).wait()

    def scale(buf):
        r = rows[buf]

        def body(i, carry):
            base = i * UNROLL
            for u in range(UNROLL):
                for c in range(EMB // LANES):
                    sl = pl.ds(c * LANES, LANES)
                    r[base + u, sl] = r[base + u, sl] * SCALE
            return carry

        lax.fori_loop(0, C // UNROLL, body, 0)

    def run_chunk(ch, buf, fire_next, wait_prev_wb):
        if wait_prev_wb:
            wait_wb(1 - buf)
        if fire_next:
            fire_gathers(ch + 1, 1 - buf)
        wait_gathers(buf)
        scale(buf)
        fire_wb(ch, buf)

    fire_gathers(0, 0)
    run_chunk(0, 0, fire_next=True, wait_prev_wb=False)

    def pair(i, carry):
        ch = 1 + 2 * i
        run_chunk(ch, 1, fire_next=True, wait_prev_wb=True)
        run_chunk(ch + 1, 0, fire_next=True, wait_prev_wb=True)
        return carry

    lax.fori_loop(0, (chunks - 2) // 2, pair, 0)

    run_chunk(chunks - 1, 1, fire_next=False, wait_prev_wb=True)
    wait_wb(1)


def kernel(tokens, table):
    b, t = tokens.shape
    n = b * t
    idx = tokens.reshape(n).astype(jnp.int32).reshape(n // IDXW, IDXW)
    mesh = plsc.VectorSubcoreMesh(core_axis_name="c", subcore_axis_name="s")
    run = functools.partial(
        pl.kernel,
        mesh=mesh,
        compiler_params=pltpu.CompilerParams(use_tc_tiling_on_sc=False),
        out_type=jax.ShapeDtypeStruct((n, EMB), jnp.float32),
        scratch_types=[
            pltpu.VMEM((n // IDXW // NW, IDXW), jnp.int32),
            pltpu.VMEM((C, EMB), jnp.float32),
            pltpu.VMEM((C, EMB), jnp.float32),
            pltpu.SemaphoreType.DMA,
            pltpu.SemaphoreType.DMA,
            pltpu.SemaphoreType.DMA,
            pltpu.SemaphoreType.DMA,
        ],
    )(_emb_body)
    out = run(idx, table)
    return out.reshape(b, t, EMB)


# speedup vs baseline: 1.1376x; 1.0267x over previous
import functools
import math

import jax
import jax.numpy as jnp
from jax import lax
from jax.experimental import pallas as pl
from jax.experimental.pallas import tpu as pltpu
from jax.experimental.pallas import tpu_sc as plsc

EMB = 64
LANES = 16
IDXW = 128
K = 5
C = K * IDXW
NC = 2
NS = 16
NW = NC * NS
SCALE = math.sqrt(EMB)
UNROLL = 8


def _emb_body(idx_hbm, table_hbm, out_hbm,
              idx_v, rows0, rows1, sg0, sg1, sw0, sw1):
    wid = lax.axis_index("s") * NC + lax.axis_index("c")
    idx_rows = idx_hbm.shape[0] // NW
    chunks = idx_rows // K
    row0 = wid * idx_rows

    rows = (rows0, rows1)
    sg = (sg0, sg1)
    sw = (sw0, sw1)

    pltpu.sync_copy(idx_hbm.at[pl.ds(row0, idx_rows)], idx_v)

    def fire_gathers(ch, buf):
        for j in range(K):
            pltpu.async_copy(
                table_hbm.at[idx_v.at[ch * K + j]],
                rows[buf].at[pl.ds(j * IDXW, IDXW)],
                sg[buf],
            )

    def wait_gathers(buf):
        pltpu.make_async_copy(out_hbm.at[pl.ds(0, C)], rows[buf], sg[buf]).wait()

    def fire_wb(ch, buf):
        pltpu.async_copy(
            rows[buf], out_hbm.at[pl.ds((row0 + ch * K) * IDXW, C)], sw[buf])

    def wait_wb(buf):
        pltpu.make_async_copy(rows[buf], out_hbm.at[pl.ds(0, C)], sw[buf]).wait()

    def scale(buf):
        r = rows[buf]

        def body(i, carry):
            base = i * UNROLL
            for u in range(UNROLL):
                for c in range(EMB // LANES):
                    sl = pl.ds(c * LANES, LANES)
                    r[base + u, sl] = r[base + u, sl] * SCALE
            return carry

        lax.fori_loop(0, C // UNROLL, body, 0)

    def run_chunk(ch, buf, fire_next, wait_prev_wb):
        if wait_prev_wb:
            wait_wb(1 - buf)
        if fire_next:
            fire_gathers(ch + 1, 1 - buf)
        wait_gathers(buf)
        scale(buf)
        fire_wb(ch, buf)

    fire_gathers(0, 0)
    run_chunk(0, 0, fire_next=True, wait_prev_wb=False)

    def pair(i, carry):
        ch = 1 + 2 * i
        run_chunk(ch, 1, fire_next=True, wait_prev_wb=True)
        run_chunk(ch + 1, 0, fire_next=True, wait_prev_wb=True)
        return carry

    lax.fori_loop(0, (chunks - 2) // 2, pair, 0)

    run_chunk(chunks - 1, 1, fire_next=False, wait_prev_wb=True)
    wait_wb(1)


def kernel(tokens, table):
    b, t = tokens.shape
    n = b * t
    idx = tokens.T.reshape(n).astype(jnp.int32).reshape(n // IDXW, IDXW)
    mesh = plsc.VectorSubcoreMesh(core_axis_name="c", subcore_axis_name="s")
    run = functools.partial(
        pl.kernel,
        mesh=mesh,
        compiler_params=pltpu.CompilerParams(use_tc_tiling_on_sc=False),
        out_type=jax.ShapeDtypeStruct((n, EMB), jnp.float32),
        scratch_types=[
            pltpu.VMEM((n // IDXW // NW, IDXW), jnp.int32),
            pltpu.VMEM((C, EMB), jnp.float32),
            pltpu.VMEM((C, EMB), jnp.float32),
            pltpu.SemaphoreType.DMA,
            pltpu.SemaphoreType.DMA,
            pltpu.SemaphoreType.DMA,
            pltpu.SemaphoreType.DMA,
        ],
    )(_emb_body)
    out = run(idx, table)
    return out.reshape(t, b, EMB).transpose(1, 0, 2)
